# baseline (speedup 1.0000x reference)
import functools

import jax
import jax.numpy as jnp
from jax import lax
from jax.experimental import pallas as pl
from jax.experimental.pallas import tpu as pltpu
from jax.experimental.pallas import tpu_sc as plsc

N = 10000
E = 320000
D = 128

NHALF = 2
E2 = E // NHALF
NC = 2
NS = 16
NW = NC * NS
EPW = E2 // NW
CHUNK = 40
NCH = EPW // CHUNK
NBUF = 5
TE = 3200


def _mesh():
    return plsc.VectorSubcoreMesh(core_axis_name="c", subcore_axis_name="s")


def _ring(drain, fire):
    for b in range(NBUF):
        fire(b, b)

    def group(g, carry):
        for b in range(NBUF):
            ch = g * NBUF + b
            drain(ch, b)
            fire(ch + NBUF, b)
        return carry

    lax.fori_loop(0, NCH // NBUF - 1, group, 0)
    g_last = NCH // NBUF - 1
    for b in range(NBUF):
        drain(g_last * NBUF + b, b)


def _gather_call(x, src, h):
    @functools.partial(
        pl.kernel,
        out_type=jax.ShapeDtypeStruct((E2 // CHUNK, CHUNK, D // 2), jnp.int32),
        mesh=_mesh(),
        scratch_types=(
            [pltpu.VMEM((EPW,), jnp.int32)]
            + [pltpu.VMEM((CHUNK, D // 2), jnp.int32) for _ in range(NBUF)]
            + [pltpu.VMEM_SHARED((N, D // 2), jnp.int32)]
            + [pltpu.SemaphoreType.DMA for _ in range(NBUF)]
        ),
    )
    def gather_kernel(x_hbm, src_hbm, xj_hbm, idx_v, b0, b1, b2, b3, b4,
                      xs_sp, s0, s1, s2, s3, s4):
        bufs = (b0, b1, b2, b3, b4)
        sems = (s0, s1, s2, s3, s4)
        s = lax.axis_index("s")
        wid = s * NC + lax.axis_index("c")
        base = wid * EPW

        @pl.when(s == 0)
        def _():
            pltpu.sync_copy(x_hbm, xs_sp)

        pltpu.sync_copy(src_hbm.at[pl.ds(h * E2 + base, EPW)], idx_v)
        plsc.subcore_barrier()

        def fire(ch, b):
            pltpu.async_copy(
                xs_sp.at[idx_v.at[pl.ds(ch * CHUNK, CHUNK)]],
                bufs[b], sems[b])

        def drain(ch, b):
            pltpu.make_async_copy(
                xs_sp.at[idx_v.at[pl.ds(ch * CHUNK, CHUNK)]],
                bufs[b], sems[b]).wait()
            pltpu.sync_copy(bufs[b], xj_hbm.at[wid * NCH + ch])

        _ring(drain, fire)

    return gather_kernel(x, src)


def _msg_body(attr_ref, xj_ref, w_ref, out_ref):
    attr = attr_ref[...].T
    p = jnp.broadcast_to(jnp.clip(attr[:, 0:1], -1.0, 1.0),
                         (TE, D)).astype(jnp.bfloat16)
    q = jnp.broadcast_to(jnp.clip(attr[:, 1:2], -1.0, 1.0),
                         (TE, D)).astype(jnp.bfloat16)
    two = jnp.bfloat16(2.0)
    one = jnp.bfloat16(1.0)
    v = xj_ref[...].reshape(TE, D // 2)
    xlo = jax.lax.bitcast_convert_type(
        jax.lax.shift_left(v, 16), jnp.float32).astype(jnp.bfloat16)
    xhi = jax.lax.bitcast_convert_type(
        jax.lax.bitwise_and(v, jnp.int32(-65536)),
        jnp.float32).astype(jnp.bfloat16)
    xj = jnp.concatenate([xlo, xhi], axis=1)
    us = (None, p, two * p * p - one)
    vs = (None, q, two * q * q - one)
    acc = None
    for a in range(3):
        xa = xj if us[a] is None else xj * us[a]
        for b in range(3):
            xs = xa if vs[b] is None else xa * vs[b]
            t = jnp.dot(xs, w_ref[3 * a + b],
                        preferred_element_type=jnp.float32)
            acc = t if acc is None else acc + t
    out_ref[...] = acc.reshape(TE // CHUNK, CHUNK, D)


def _msg_call(attr8, x_j, w9, h):
    off = h * (E2 // TE)
    return pl.pallas_call(
        _msg_body,
        grid=(E2 // TE,),
        in_specs=[
            pl.BlockSpec((2, TE), lambda i: (0, i + off)),
            pl.BlockSpec((TE // CHUNK, CHUNK, D // 2), lambda i: (i, 0, 0)),
            pl.BlockSpec((9, D, D), lambda i: (0, 0, 0)),
        ],
        out_specs=pl.BlockSpec((TE // CHUNK, CHUNK, D), lambda i: (i, 0, 0)),
        out_shape=jax.ShapeDtypeStruct((E2 // CHUNK, CHUNK, D), jnp.float32),
    )(attr8, x_j, w9)


def _scatter_call(msg, dst4, zeros, h):
    @functools.partial(
        pl.kernel,
        out_type=jax.ShapeDtypeStruct((NC, N, D), jnp.float32),
        mesh=_mesh(),
        scratch_types=(
            [pltpu.VMEM((NCH, CHUNK), jnp.int32)]
            + [pltpu.VMEM((CHUNK, D), jnp.float32) for _ in range(NBUF)]
            + [pltpu.VMEM_SHARED((N, D), jnp.float32)]
            + [pltpu.SemaphoreType.DMA for _ in range(NBUF)]
        ),
    )
    def scatter_kernel(msg_hbm, dst_hbm, zeros_hbm, part_hbm, idx_v,
                       b0, b1, b2, b3, b4, acc, s0, s1, s2, s3, s4):
        bufs = (b0, b1, b2, b3, b4)
        sems = (s0, s1, s2, s3, s4)
        c = lax.axis_index("c")
        s = lax.axis_index("s")
        wid = s * NC + c
        base = wid * EPW

        @pl.when(s == 0)
        def _():
            pltpu.sync_copy(zeros_hbm, acc)

        pltpu.sync_copy(dst_hbm.at[h, wid], idx_v)
        plsc.subcore_barrier()

        def fire(ch, b):
            pltpu.async_copy(msg_hbm.at[wid * NCH + ch], bufs[b], sems[b])

        def drain(ch, b):
            pltpu.make_async_copy(
                msg_hbm.at[wid * NCH + ch], bufs[b], sems[b]).wait()
            pltpu.sync_copy(bufs[b], acc.at[idx_v.at[ch]], add=True)

        _ring(drain, fire)
        plsc.subcore_barrier()
        @pl.when(s == 0)
        def _():
            pltpu.sync_copy(acc, part_hbm.at[c])

    return scatter_kernel(msg, dst4, zeros)


def _add_body(pa_ref, pb_ref, o_ref):
    o_ref[...] = (pa_ref[0] + pa_ref[1]) + (pb_ref[0] + pb_ref[1])


def _add_call(part_a, part_b):
    spec = pl.BlockSpec((NC, N // 10, D), lambda i: (0, i, 0))
    return pl.pallas_call(
        _add_body,
        grid=(10,),
        in_specs=[spec, spec],
        out_specs=pl.BlockSpec((N // 10, D), lambda i: (i, 0)),
        out_shape=jax.ShapeDtypeStruct((N, D), jnp.float32),
    )(part_a, part_b)


def kernel(x, edge_index, edge_attr, weight):
    w9 = weight.reshape(9, D, D).astype(jnp.bfloat16)
    xb = x.astype(jnp.bfloat16)
    lo = jax.lax.bitcast_convert_type(xb[:, :D // 2], jnp.uint16)
    hi = jax.lax.bitcast_convert_type(xb[:, D // 2:], jnp.uint16)
    x32 = jax.lax.bitcast_convert_type(
        lo.astype(jnp.uint32) | (hi.astype(jnp.uint32) << 16), jnp.int32)
    attr8 = edge_attr.T
    src = edge_index[1]
    dst4 = edge_index[0].reshape(NHALF, NW, NCH, CHUNK)
    zeros = jnp.zeros((N, D), jnp.float32)
    parts = []
    for h in range(NHALF):
        x_j = _gather_call(x32, src, h)
        msg = _msg_call(attr8, x_j, w9, h)
        parts.append(_scatter_call(msg, dst4, zeros, h))
    return _add_call(*parts)


# speedup vs baseline: 1.0712x; 1.0712x over previous
import functools

import jax
import jax.numpy as jnp
from jax import lax
from jax.experimental import pallas as pl
from jax.experimental.pallas import tpu as pltpu
from jax.experimental.pallas import tpu_sc as plsc

N = 10000
E = 320000
D = 128

NHALF = 2
E2 = E // NHALF
NC = 2
NS = 16
NW = NC * NS
EPW = E2 // NW
CHUNK = 40
NCH = EPW // CHUNK
NBUF = 5
TE = 3200


def _mesh():
    return plsc.VectorSubcoreMesh(core_axis_name="c", subcore_axis_name="s")


def _ring(drain, fire):
    for b in range(NBUF):
        fire(b, b)

    def group(g, carry):
        for b in range(NBUF):
            ch = g * NBUF + b
            drain(ch, b)
            fire(ch + NBUF, b)
        return carry

    lax.fori_loop(0, NCH // NBUF - 1, group, 0)
    g_last = NCH // NBUF - 1
    for b in range(NBUF):
        drain(g_last * NBUF + b, b)


def _gather_call(x, src, h):
    @functools.partial(
        pl.kernel,
        out_type=jax.ShapeDtypeStruct((E2 // CHUNK, CHUNK, D), jnp.float32),
        mesh=_mesh(),
        scratch_types=(
            [pltpu.VMEM((EPW,), jnp.int32)]
            + [pltpu.VMEM((CHUNK, D), jnp.float32) for _ in range(NBUF)]
            + [pltpu.VMEM_SHARED((N, D), jnp.float32)]
            + [pltpu.SemaphoreType.DMA for _ in range(NBUF)]
        ),
    )
    def gather_kernel(x_hbm, src_hbm, xj_hbm, idx_v, b0, b1, b2, b3, b4,
                      xs_sp, s0, s1, s2, s3, s4):
        bufs = (b0, b1, b2, b3, b4)
        sems = (s0, s1, s2, s3, s4)
        s = lax.axis_index("s")
        wid = s * NC + lax.axis_index("c")
        base = wid * EPW

        @pl.when(s == 0)
        def _():
            pltpu.sync_copy(x_hbm, xs_sp)

        pltpu.sync_copy(src_hbm.at[pl.ds(h * E2 + base, EPW)], idx_v)
        plsc.subcore_barrier()

        def fire(ch, b):
            pltpu.async_copy(
                xs_sp.at[idx_v.at[pl.ds(ch * CHUNK, CHUNK)]],
                bufs[b], sems[b])

        def drain(ch, b):
            pltpu.make_async_copy(
                xs_sp.at[idx_v.at[pl.ds(ch * CHUNK, CHUNK)]],
                bufs[b], sems[b]).wait()
            pltpu.sync_copy(bufs[b], xj_hbm.at[wid * NCH + ch])

        _ring(drain, fire)

    return gather_kernel(x, src)


def _msg_body(attr_ref, xj_ref, w_ref, out_ref):
    attr = attr_ref[...].T
    p = jnp.broadcast_to(jnp.clip(attr[:, 0:1], -1.0, 1.0),
                         (TE, D)).astype(jnp.bfloat16)
    q = jnp.broadcast_to(jnp.clip(attr[:, 1:2], -1.0, 1.0),
                         (TE, D)).astype(jnp.bfloat16)
    two = jnp.bfloat16(2.0)
    one = jnp.bfloat16(1.0)
    xj = xj_ref[...].reshape(TE, D).astype(jnp.bfloat16)
    us = (None, p, two * p * p - one)
    vs = (None, q, two * q * q - one)
    acc = None
    for a in range(3):
        xa = xj if us[a] is None else xj * us[a]
        for b in range(3):
            xs = xa if vs[b] is None else xa * vs[b]
            t = jnp.dot(xs, w_ref[3 * a + b],
                        preferred_element_type=jnp.float32)
            acc = t if acc is None else acc + t
    out_ref[...] = acc.reshape(TE // CHUNK, CHUNK, D)


def _msg_call(attr8, x_j, w9, h):
    off = h * (E2 // TE)
    return pl.pallas_call(
        _msg_body,
        grid=(E2 // TE,),
        in_specs=[
            pl.BlockSpec((2, TE), lambda i: (0, i + off)),
            pl.BlockSpec((TE // CHUNK, CHUNK, D), lambda i: (i, 0, 0)),
            pl.BlockSpec((9, D, D), lambda i: (0, 0, 0)),
        ],
        out_specs=pl.BlockSpec((TE // CHUNK, CHUNK, D), lambda i: (i, 0, 0)),
        out_shape=jax.ShapeDtypeStruct((E2 // CHUNK, CHUNK, D), jnp.float32),
        compiler_params=pltpu.CompilerParams(
            dimension_semantics=("parallel",)),
    )(attr8, x_j, w9)


def _scatter_call(msg, dst4, zeros, h):
    @functools.partial(
        pl.kernel,
        out_type=jax.ShapeDtypeStruct((NC, N, D), jnp.float32),
        mesh=_mesh(),
        scratch_types=(
            [pltpu.VMEM((NCH, CHUNK), jnp.int32)]
            + [pltpu.VMEM((CHUNK, D), jnp.float32) for _ in range(NBUF)]
            + [pltpu.VMEM_SHARED((N, D), jnp.float32)]
            + [pltpu.SemaphoreType.DMA for _ in range(NBUF)]
        ),
    )
    def scatter_kernel(msg_hbm, dst_hbm, zeros_hbm, part_hbm, idx_v,
                       b0, b1, b2, b3, b4, acc, s0, s1, s2, s3, s4):
        bufs = (b0, b1, b2, b3, b4)
        sems = (s0, s1, s2, s3, s4)
        c = lax.axis_index("c")
        s = lax.axis_index("s")
        wid = s * NC + c
        base = wid * EPW

        @pl.when(s == 0)
        def _():
            pltpu.sync_copy(zeros_hbm, acc)

        pltpu.sync_copy(dst_hbm.at[h, wid], idx_v)
        plsc.subcore_barrier()

        def fire(ch, b):
            pltpu.async_copy(msg_hbm.at[wid * NCH + ch], bufs[b], sems[b])

        def drain(ch, b):
            pltpu.make_async_copy(
                msg_hbm.at[wid * NCH + ch], bufs[b], sems[b]).wait()
            pltpu.sync_copy(bufs[b], acc.at[idx_v.at[ch]], add=True)

        _ring(drain, fire)
        plsc.subcore_barrier()
        @pl.when(s == 0)
        def _():
            pltpu.sync_copy(acc, part_hbm.at[c])

    return scatter_kernel(msg, dst4, zeros)


def _add_body(pa_ref, pb_ref, o_ref):
    o_ref[...] = (pa_ref[0] + pa_ref[1]) + (pb_ref[0] + pb_ref[1])


def _add_call(part_a, part_b):
    spec = pl.BlockSpec((NC, N // 10, D), lambda i: (0, i, 0))
    return pl.pallas_call(
        _add_body,
        grid=(10,),
        in_specs=[spec, spec],
        out_specs=pl.BlockSpec((N // 10, D), lambda i: (i, 0)),
        out_shape=jax.ShapeDtypeStruct((N, D), jnp.float32),
    )(part_a, part_b)


def kernel(x, edge_index, edge_attr, weight):
    w9 = weight.reshape(9, D, D).astype(jnp.bfloat16)
    attr8 = edge_attr.T
    src = edge_index[1]
    dst4 = edge_index[0].reshape(NHALF, NW, NCH, CHUNK)
    zeros = jnp.zeros((N, D), jnp.float32)
    parts = []
    for h in range(NHALF):
        x_j = _gather_call(x, src, h)
        msg = _msg_call(attr8, x_j, w9, h)
        parts.append(_scatter_call(msg, dst4, zeros, h))
    return _add_call(*parts)


# speedup vs baseline: 1.1273x; 1.0523x over previous
import functools

import jax
import jax.numpy as jnp
from jax import lax
from jax.experimental import pallas as pl
from jax.experimental.pallas import tpu as pltpu
from jax.experimental.pallas import tpu_sc as plsc

N = 10000
E = 320000
D = 128

NHALF = 2
E2 = E // NHALF
NC = 2
NS = 16
NW = NC * NS
EPW = E2 // NW
CHUNK = 40
NCH = EPW // CHUNK
NBUF = 5
TE = 6400


def _mesh():
    return plsc.VectorSubcoreMesh(core_axis_name="c", subcore_axis_name="s")


def _ring(drain, fire):
    for b in range(NBUF):
        fire(b, b)

    def group(g, carry):
        for b in range(NBUF):
            ch = g * NBUF + b
            drain(ch, b)
            fire(ch + NBUF, b)
        return carry

    lax.fori_loop(0, NCH // NBUF - 1, group, 0)
    g_last = NCH // NBUF - 1
    for b in range(NBUF):
        drain(g_last * NBUF + b, b)


def _gather_call(x, src, h):
    @functools.partial(
        pl.kernel,
        out_type=jax.ShapeDtypeStruct((E2 // CHUNK, CHUNK, D), jnp.float32),
        mesh=_mesh(),
        scratch_types=(
            [pltpu.VMEM((EPW,), jnp.int32)]
            + [pltpu.VMEM((CHUNK, D), jnp.float32) for _ in range(NBUF)]
            + [pltpu.VMEM_SHARED((N, D), jnp.float32)]
            + [pltpu.SemaphoreType.DMA for _ in range(NBUF)]
        ),
    )
    def gather_kernel(x_hbm, src_hbm, xj_hbm, idx_v, b0, b1, b2, b3, b4,
                      xs_sp, s0, s1, s2, s3, s4):
        bufs = (b0, b1, b2, b3, b4)
        sems = (s0, s1, s2, s3, s4)
        s = lax.axis_index("s")
        wid = s * NC + lax.axis_index("c")
        base = wid * EPW

        @pl.when(s == 0)
        def _():
            pltpu.sync_copy(x_hbm, xs_sp)

        pltpu.sync_copy(src_hbm.at[pl.ds(h * E2 + base, EPW)], idx_v)
        plsc.subcore_barrier()

        def fire(ch, b):
            pltpu.async_copy(
                xs_sp.at[idx_v.at[pl.ds(ch * CHUNK, CHUNK)]],
                bufs[b], sems[b])

        def drain(ch, b):
            pltpu.make_async_copy(
                xs_sp.at[idx_v.at[pl.ds(ch * CHUNK, CHUNK)]],
                bufs[b], sems[b]).wait()
            pltpu.sync_copy(bufs[b], xj_hbm.at[wid * NCH + ch])

        _ring(drain, fire)

    return gather_kernel(x, src)


def _msg_body(attr_ref, xj_ref, w_ref, out_ref):
    attr = attr_ref[...].T
    p = jnp.broadcast_to(jnp.clip(attr[:, 0:1], -1.0, 1.0),
                         (TE, D)).astype(jnp.bfloat16)
    q = jnp.broadcast_to(jnp.clip(attr[:, 1:2], -1.0, 1.0),
                         (TE, D)).astype(jnp.bfloat16)
    two = jnp.bfloat16(2.0)
    one = jnp.bfloat16(1.0)
    xj = xj_ref[...].reshape(TE, D).astype(jnp.bfloat16)
    us = (None, p, two * p * p - one)
    vs = (None, q, two * q * q - one)
    acc = None
    for a in range(3):
        xa = xj if us[a] is None else xj * us[a]
        for b in range(3):
            xs = xa if vs[b] is None else xa * vs[b]
            t = jnp.dot(xs, w_ref[3 * a + b],
                        preferred_element_type=jnp.float32)
            acc = t if acc is None else acc + t
    out_ref[...] = acc.reshape(TE // CHUNK, CHUNK, D)


def _msg_call(attr8, x_j, w9, h):
    off = h * (E2 // TE)
    return pl.pallas_call(
        _msg_body,
        grid=(E2 // TE,),
        in_specs=[
            pl.BlockSpec((2, TE), lambda i: (0, i + off)),
            pl.BlockSpec((TE // CHUNK, CHUNK, D), lambda i: (i, 0, 0)),
            pl.BlockSpec((9, D, D), lambda i: (0, 0, 0)),
        ],
        out_specs=pl.BlockSpec((TE // CHUNK, CHUNK, D), lambda i: (i, 0, 0)),
        out_shape=jax.ShapeDtypeStruct((E2 // CHUNK, CHUNK, D), jnp.float32),
        compiler_params=pltpu.CompilerParams(
            dimension_semantics=("parallel",)),
    )(attr8, x_j, w9)


def _scatter_call(msg, dst4, zeros, h):
    @functools.partial(
        pl.kernel,
        out_type=jax.ShapeDtypeStruct((NC, N, D), jnp.float32),
        mesh=_mesh(),
        scratch_types=(
            [pltpu.VMEM((NCH, CHUNK), jnp.int32)]
            + [pltpu.VMEM((CHUNK, D), jnp.float32) for _ in range(NBUF)]
            + [pltpu.VMEM_SHARED((N, D), jnp.float32)]
            + [pltpu.SemaphoreType.DMA for _ in range(NBUF)]
        ),
    )
    def scatter_kernel(msg_hbm, dst_hbm, zeros_hbm, part_hbm, idx_v,
                       b0, b1, b2, b3, b4, acc, s0, s1, s2, s3, s4):
        bufs = (b0, b1, b2, b3, b4)
        sems = (s0, s1, s2, s3, s4)
        c = lax.axis_index("c")
        s = lax.axis_index("s")
        wid = s * NC + c
        base = wid * EPW

        @pl.when(s == 0)
        def _():
            pltpu.sync_copy(zeros_hbm, acc)

        pltpu.sync_copy(dst_hbm.at[h, wid], idx_v)
        plsc.subcore_barrier()

        def fire(ch, b):
            pltpu.async_copy(msg_hbm.at[wid * NCH + ch], bufs[b], sems[b])

        def drain(ch, b):
            pltpu.make_async_copy(
                msg_hbm.at[wid * NCH + ch], bufs[b], sems[b]).wait()
            pltpu.sync_copy(bufs[b], acc.at[idx_v.at[ch]], add=True)

        _ring(drain, fire)
        plsc.subcore_barrier()
        @pl.when(s == 0)
        def _():
            pltpu.sync_copy(acc, part_hbm.at[c])

    return scatter_kernel(msg, dst4, zeros)


def _add_body(pa_ref, pb_ref, o_ref):
    o_ref[...] = (pa_ref[0] + pa_ref[1]) + (pb_ref[0] + pb_ref[1])


def _add_call(part_a, part_b):
    spec = pl.BlockSpec((NC, N // 10, D), lambda i: (0, i, 0))
    return pl.pallas_call(
        _add_body,
        grid=(10,),
        in_specs=[spec, spec],
        out_specs=pl.BlockSpec((N // 10, D), lambda i: (i, 0)),
        out_shape=jax.ShapeDtypeStruct((N, D), jnp.float32),
    )(part_a, part_b)


def kernel(x, edge_index, edge_attr, weight):
    w9 = weight.reshape(9, D, D).astype(jnp.bfloat16)
    attr8 = edge_attr.T
    src = edge_index[1]
    dst4 = edge_index[0].reshape(NHALF, NW, NCH, CHUNK)
    zeros = jnp.zeros((N, D), jnp.float32)
    parts = []
    for h in range(NHALF):
        x_j = _gather_call(x, src, h)
        msg = _msg_call(attr8, x_j, w9, h)
        parts.append(_scatter_call(msg, dst4, zeros, h))
    return _add_call(*parts)
